# baseline (speedup 1.0000x reference)
import functools

import jax
import jax.numpy as jnp
from jax import lax
from jax.experimental import pallas as pl
from jax.experimental.pallas import tpu as pltpu
from jax.experimental.pallas import tpu_sc as plsc

_ROWS = 16384
_COLS = 200
_DIM = 32
_L = 16
_NW = 32
_STRIPE = _ROWS // _NW
_SPLIT = 144
_BLOCK = 512


def _tc_body(idx_ref, t0_ref, dt_ref, out_ref):
    w = idx_ref[...].astype(jnp.float32)[:, None, :]
    t0 = t0_ref[...][None, :, :]
    dt = dt_ref[...][None, :, :]
    out_ref[...] = t0 + w * dt


def _sc_body(idx_hbm, t0_hbm, dt_hbm, out_hbm,
             idx_v, w_v, t0_v, dt_v, o_v, isem0, isem1, osem0, osem1):
    wid = lax.axis_index("s") * 2 + lax.axis_index("c")
    base = wid * _STRIPE
    pltpu.sync_copy(t0_hbm, t0_v)
    pltpu.sync_copy(dt_hbm, dt_v)
    isems = (isem0, isem1)
    osems = (osem0, osem1)
    n = _COLS - _SPLIT

    def idx_copy(j, b):
        return pltpu.make_async_copy(
            idx_hbm.at[_SPLIT + j, pl.ds(base, _STRIPE)], idx_v.at[b], isems[b])

    def out_copy(j, b):
        return pltpu.make_async_copy(
            o_v.at[b], out_hbm.at[j, :, pl.ds(base, _STRIPE)], osems[b])

    idx_copy(0, 0).start()
    idx_copy(1, 1).start()

    @pl.loop(0, n // 2)
    def _(jj):
        for b in range(2):
            j = jj * 2 + b
            idx_copy(j, b).wait()
            for c in range(_STRIPE // _L):
                sl = pl.ds(c * _L, _L)
                w_v[sl] = idx_v[b, sl].astype(jnp.float32)

            @pl.when(j + 2 < n)
            def _():
                idx_copy(j + 2, b).start()

            @pl.when(jj > 0)
            def _():
                out_copy(j - 2, b).wait()

            for k in range(_DIM):
                t0 = t0_v[k]
                dt = dt_v[k]
                for c in range(_STRIPE // _L):
                    sl = pl.ds(c * _L, _L)
                    o_v[b, k, sl] = t0 + w_v[sl] * dt
            out_copy(j, b).start()

    out_copy(n - 2, 0).wait()
    out_copy(n - 1, 1).wait()


def kernel(inputs, table):
    rows, cols = inputs.shape
    dim = table.shape[1]
    idx_t = inputs.T
    t0_col = table[0].reshape(dim, 1)
    dt_col = (table[1] - table[0]).reshape(dim, 1)
    t0_rep = jnp.broadcast_to(t0_col, (dim, _L))
    dt_rep = jnp.broadcast_to(dt_col, (dim, _L))

    tc_out = pl.pallas_call(
        _tc_body,
        grid=(rows // _BLOCK,),
        in_specs=[
            pl.BlockSpec((_SPLIT, _BLOCK), lambda i: (0, i)),
            pl.BlockSpec((dim, 1), lambda i: (0, 0)),
            pl.BlockSpec((dim, 1), lambda i: (0, 0)),
        ],
        out_specs=pl.BlockSpec((_SPLIT, dim, _BLOCK), lambda i: (0, 0, i)),
        out_shape=jax.ShapeDtypeStruct((_SPLIT, dim, rows), jnp.float32),
    )(idx_t, t0_col, dt_col)

    mesh = plsc.VectorSubcoreMesh(core_axis_name="c", subcore_axis_name="s")
    sck = pl.kernel(
        _sc_body,
        out_type=jax.ShapeDtypeStruct((cols - _SPLIT, dim, rows), jnp.float32),
        mesh=mesh,
        scratch_types=[
            pltpu.VMEM((2, _STRIPE), jnp.int32),
            pltpu.VMEM((_STRIPE,), jnp.float32),
            pltpu.VMEM((dim, _L), jnp.float32),
            pltpu.VMEM((dim, _L), jnp.float32),
            pltpu.VMEM((2, dim, _STRIPE), jnp.float32),
            pltpu.SemaphoreType.DMA,
            pltpu.SemaphoreType.DMA,
            pltpu.SemaphoreType.DMA,
            pltpu.SemaphoreType.DMA,
        ],
        compiler_params=pltpu.CompilerParams(use_tc_tiling_on_sc=True),
    )
    sc_out = sck(idx_t, t0_rep, dt_rep)

    out_t = jnp.concatenate([tc_out, sc_out], axis=0)
    return out_t.transpose(2, 0, 1)


# speedup vs baseline: 3.1401x; 3.1401x over previous
import jax
import jax.numpy as jnp
from jax.experimental import pallas as pl
from jax.experimental.pallas import tpu as pltpu

_BLOCK = 512


def _embed_block(idx_ref, t0_ref, dt_ref, out_ref):
    w = idx_ref[...].astype(jnp.float32)[:, None, :]
    t0 = t0_ref[...][None, :, :]
    dt = dt_ref[...][None, :, :]
    out_ref[...] = t0 + w * dt


def kernel(inputs, table):
    rows, cols = inputs.shape
    dim = table.shape[1]
    idx_t = inputs.T
    t0 = table[0].reshape(dim, 1)
    dt = (table[1] - table[0]).reshape(dim, 1)
    grid = (rows // _BLOCK,)
    out_t = pl.pallas_call(
        _embed_block,
        grid=grid,
        in_specs=[
            pl.BlockSpec((cols, _BLOCK), lambda i: (0, i)),
            pl.BlockSpec((dim, 1), lambda i: (0, 0)),
            pl.BlockSpec((dim, 1), lambda i: (0, 0)),
        ],
        out_specs=pl.BlockSpec((cols, dim, _BLOCK), lambda i: (0, 0, i)),
        out_shape=jax.ShapeDtypeStruct((cols, dim, rows), jnp.float32),
    )(idx_t, t0, dt)
    return out_t.transpose(2, 0, 1)
